# Initial kernel scaffold; baseline (speedup 1.0000x reference)
import functools

import jax
import jax.numpy as jnp
from jax import lax
from jax.experimental import pallas as pl
from jax.experimental.pallas import tpu as pltpu
from jax.experimental.pallas import tpu_sc as plsc

BATCH = 16384
HIST = 200
EMB = 16
N = BATCH * HIST

NUM_CORES = 2
NUM_SUBCORES = 16
NW = NUM_CORES * NUM_SUBCORES
PER_W = N // NW
CHUNK = 2048
N_CHUNKS = PER_W // CHUNK


def _body(x_hbm, table_hbm, out_hbm, idx_v, rows_v, sem):
    wid = lax.axis_index("s") * NUM_CORES + lax.axis_index("c")
    base = wid * PER_W

    def step(g, carry):
        off = base + g * CHUNK
        pltpu.sync_copy(x_hbm.at[pl.ds(off, CHUNK)], idx_v)
        pltpu.async_copy(table_hbm.at[idx_v], rows_v, sem).wait()
        pltpu.sync_copy(rows_v, out_hbm.at[pl.ds(off, CHUNK)])
        return carry

    lax.fori_loop(0, N_CHUNKS, step, 0, unroll=False)


@jax.jit
def _lookup(x_flat, table):
    mesh = plsc.VectorSubcoreMesh(core_axis_name="c", subcore_axis_name="s")
    return pl.kernel(
        _body,
        out_type=jax.ShapeDtypeStruct((N, EMB), jnp.float32),
        mesh=mesh,
        scratch_types=[
            pltpu.VMEM((CHUNK,), jnp.int32),
            pltpu.VMEM((CHUNK, EMB), jnp.float32),
            pltpu.SemaphoreType.DMA,
        ],
    )(x_flat, table)


def kernel(x, table):
    x_flat = x.reshape(N).astype(jnp.int32)
    out = _lookup(x_flat, table)
    return out.reshape(BATCH, HIST, EMB)


# speedup vs baseline: 2.4907x; 2.4907x over previous
import functools

import jax
import jax.numpy as jnp
from jax import lax
from jax.experimental import pallas as pl
from jax.experimental.pallas import tpu as pltpu
from jax.experimental.pallas import tpu_sc as plsc

BATCH = 16384
HIST = 200
EMB = 16
N = BATCH * HIST

NUM_CORES = 2
NUM_SUBCORES = 16
NW = NUM_CORES * NUM_SUBCORES
PER_W = N // NW
CHUNK = 2048
N_CHUNKS = PER_W // CHUNK


def _body(x_hbm, table_hbm, out_hbm, idx_v, rows_v, sem):
    wid = lax.axis_index("s") * NUM_CORES + lax.axis_index("c")
    base = wid * PER_W

    def step(g, carry):
        off = base + g * CHUNK
        pltpu.sync_copy(x_hbm.at[pl.ds(off, CHUNK)], idx_v)
        pltpu.async_copy(table_hbm.at[idx_v], rows_v, sem).wait()
        pltpu.sync_copy(rows_v, out_hbm.at[pl.ds(off, CHUNK)])
        return carry

    lax.fori_loop(0, N_CHUNKS, step, 0, unroll=False)


@jax.jit
def _lookup(x_flat, table):
    mesh = plsc.VectorSubcoreMesh(core_axis_name="c", subcore_axis_name="s")
    return pl.kernel(
        _body,
        out_type=jax.ShapeDtypeStruct((N, EMB), jnp.float32),
        mesh=mesh,
        scratch_types=[
            pltpu.VMEM((CHUNK,), jnp.int32),
            pltpu.VMEM((CHUNK, EMB), jnp.float32),
            pltpu.SemaphoreType.DMA,
        ],
        compiler_params=pltpu.CompilerParams(use_tc_tiling_on_sc=False),
    )(x_flat, table)


def kernel(x, table):
    x_flat = x.reshape(N).astype(jnp.int32)
    out = _lookup(x_flat, table)
    return out.reshape(BATCH, HIST, EMB)


# speedup vs baseline: 2.5714x; 1.0324x over previous
import jax
import jax.numpy as jnp
from jax import lax
from jax.experimental import pallas as pl
from jax.experimental.pallas import tpu as pltpu
from jax.experimental.pallas import tpu_sc as plsc

BATCH = 16384
HIST = 200
EMB = 16
N = BATCH * HIST

NUM_CORES = 2
NUM_SUBCORES = 16
NW = NUM_CORES * NUM_SUBCORES
PER_W = N // NW
CHUNK = 2048
NC = PER_W // CHUNK
PAIRS = NC // 2


def _body(x_hbm, table_hbm, out_hbm, idx_v, rows_v,
          ix_sem0, ix_sem1, g_sem0, g_sem1, st_sem0, st_sem1):
    wid = lax.axis_index("s") * NUM_CORES + lax.axis_index("c")
    base = wid * PER_W
    ix_sems = (ix_sem0, ix_sem1)
    g_sems = (g_sem0, g_sem1)
    st_sems = (st_sem0, st_sem1)

    def idx_copy(b, c):
        return pltpu.make_async_copy(
            x_hbm.at[pl.ds(base + c * CHUNK, CHUNK)], idx_v.at[b], ix_sems[b])

    def gather_copy(b):
        return pltpu.make_async_copy(
            table_hbm.at[idx_v.at[b]], rows_v.at[b], g_sems[b])

    def store_copy(b, c):
        return pltpu.make_async_copy(
            rows_v.at[b], out_hbm.at[pl.ds(base + c * CHUNK, CHUNK)],
            st_sems[b])

    def slot(b, c, first, last, head=False):
        if not first:
            store_copy(b, c - 2).wait()
        idx_copy(b, c).wait()
        gather_copy(b).start()
        if not head:
            gather_copy(1 - b).wait()
            store_copy(1 - b, c - 1).start()
        if not last:
            idx_copy(1 - b, c + 1).start()

    idx_copy(0, 0).start()
    slot(0, 0, first=True, last=False, head=True)
    slot(1, 1, first=True, last=False)

    def pair(t, carry):
        c0 = t * 2
        slot(0, c0, first=False, last=False)
        slot(1, c0 + 1, first=False, last=False)
        return carry

    lax.fori_loop(1, PAIRS - 1, pair, 0, unroll=False)

    c0 = NC - 2
    slot(0, c0, first=False, last=False)
    slot(1, c0 + 1, first=False, last=True)
    gather_copy(1).wait()
    store_copy(1, NC - 1).start()
    store_copy(0, NC - 2).wait()
    store_copy(1, NC - 1).wait()


@jax.jit
def _lookup(x_flat, table):
    mesh = plsc.VectorSubcoreMesh(core_axis_name="c", subcore_axis_name="s")
    return pl.kernel(
        _body,
        out_type=jax.ShapeDtypeStruct((N, EMB), jnp.float32),
        mesh=mesh,
        scratch_types=[
            pltpu.VMEM((2, CHUNK), jnp.int32),
            pltpu.VMEM((2, CHUNK, EMB), jnp.float32),
            pltpu.SemaphoreType.DMA,
            pltpu.SemaphoreType.DMA,
            pltpu.SemaphoreType.DMA,
            pltpu.SemaphoreType.DMA,
            pltpu.SemaphoreType.DMA,
            pltpu.SemaphoreType.DMA,
        ],
        compiler_params=pltpu.CompilerParams(use_tc_tiling_on_sc=False),
    )(x_flat, table)


def kernel(x, table):
    x_flat = x.reshape(N).astype(jnp.int32)
    out = _lookup(x_flat, table)
    return out.reshape(BATCH, HIST, EMB)
